# Initial kernel scaffold; baseline (speedup 1.0000x reference)
import jax
import jax.numpy as jnp
from jax.experimental import pallas as pl


def kernel(x, edge_index, l0_w1, l0_b1, l0_g, l0_beta, l0_w2, l0_b2, l1_w1, l1_b1, l1_g, l1_beta, l1_w2, l1_b2):
    raise NotImplementedError("write your pallas kernel here")


# speedup vs baseline: 4.6543x; 4.6543x over previous
import functools

import jax
import jax.numpy as jnp
from jax import lax
from jax.experimental import pallas as pl
from jax.experimental.pallas import tpu as pltpu
from jax.experimental.pallas import tpu_sc as plsc

_N = 10000
_E = 320000
_D = 128
_NC = 2
_NS = 16
_NW = _NC * _NS
_EPW = _E // _NW
_CHUNK = 80
_NCHUNK = _EPW // _CHUNK
_NP = 10240
_RPT = _NP // _NS


def _mp_body(x_hbm, src_hbm, dst_hbm, zero_hbm, out_hbm,
             src_v, dst_v, rows_v, acc_sh, sem):
    cid = lax.axis_index("c")
    sid = lax.axis_index("s")
    wid = cid * _NS + sid
    r0 = sid * _RPT
    pltpu.sync_copy(zero_hbm.at[pl.ds(r0, _RPT)], acc_sh.at[pl.ds(r0, _RPT)])
    plsc.subcore_barrier()

    base = wid * _EPW

    def chunk_body(j, carry):
        off = base + j * _CHUNK
        pltpu.sync_copy(src_hbm.at[pl.ds(off, _CHUNK)], src_v)
        pltpu.sync_copy(dst_hbm.at[pl.ds(off, _CHUNK)], dst_v)
        pltpu.async_copy(x_hbm.at[src_v], rows_v, sem).wait()
        pltpu.sync_copy(rows_v, acc_sh.at[dst_v], add=True)
        return carry

    lax.fori_loop(0, _NCHUNK, chunk_body, 0)
    plsc.subcore_barrier()
    pltpu.sync_copy(acc_sh.at[pl.ds(r0, _RPT)],
                    out_hbm.at[pl.ds(cid * _NP + r0, _RPT)])


_mp = functools.partial(
    pl.kernel,
    out_type=jax.ShapeDtypeStruct((2 * _NP, _D), jnp.float32),
    mesh=plsc.VectorSubcoreMesh(core_axis_name="c", subcore_axis_name="s"),
    scratch_types=[
        pltpu.VMEM((_CHUNK,), jnp.int32),
        pltpu.VMEM((_CHUNK,), jnp.int32),
        pltpu.VMEM((_CHUNK, _D), jnp.float32),
        pltpu.VMEM_SHARED((_NP, _D), jnp.float32),
        pltpu.SemaphoreType.DMA,
    ],
)(_mp_body)


def _dense_body(h_ref, p_ref, w1_ref, b1_ref, g_ref, bt_ref, w2_ref, b2_ref,
                o_ref, *, relu_out):
    h = h_ref[...] + p_ref[:_N] + p_ref[_NP:_NP + _N]
    y = jnp.dot(h, w1_ref[...], preferred_element_type=jnp.float32) + b1_ref[...]
    mu = jnp.mean(y, axis=0, keepdims=True)
    d = y - mu
    var = jnp.mean(d * d, axis=0, keepdims=True)
    r = jnp.maximum(d * lax.rsqrt(var + 1e-5) * g_ref[...] + bt_ref[...], 0.0)
    o = jnp.dot(r, w2_ref[...], preferred_element_type=jnp.float32) + b2_ref[...]
    if relu_out:
        o = jnp.maximum(o, 0.0)
    o_ref[...] = o


def _dense(h, p, w1, b1, g, bt, w2, b2, relu_out):
    return pl.pallas_call(
        functools.partial(_dense_body, relu_out=relu_out),
        out_shape=jax.ShapeDtypeStruct((_N, _D), jnp.float32),
    )(h, p, w1, b1.reshape(1, _D), g.reshape(1, _D), bt.reshape(1, _D),
      w2, b2.reshape(1, _D))


def kernel(x, edge_index, l0_w1, l0_b1, l0_g, l0_beta, l0_w2, l0_b2,
           l1_w1, l1_b1, l1_g, l1_beta, l1_w2, l1_b2):
    src = edge_index[0]
    dst = edge_index[1]
    zeros = jnp.zeros((_NP, _D), jnp.float32)
    p = _mp(x, src, dst, zeros)
    h0 = _dense(x, p, l0_w1, l0_b1, l0_g, l0_beta, l0_w2, l0_b2, relu_out=True)
    p = _mp(h0, src, dst, zeros)
    return _dense(h0, p, l1_w1, l1_b1, l1_g, l1_beta, l1_w2, l1_b2,
                  relu_out=False)


# speedup vs baseline: 10.7887x; 2.3180x over previous
import functools

import jax
import jax.numpy as jnp
from jax import lax
from jax.experimental import pallas as pl
from jax.experimental.pallas import tpu as pltpu
from jax.experimental.pallas import tpu_sc as plsc

_N = 10000
_E = 320000
_D = 128
_NC = 2
_NS = 16
_NW = _NC * _NS
_EPW = _E // _NW
_CHUNK = 80
_NCHUNK = _EPW // _CHUNK
_NB = 2
_NBI = 4
_NP = 10112
_RPT = _NP // _NS


def _mp_body(x_hbm, src_hbm, dst_hbm, zero_hbm, out_hbm,
             srcs_v, dsts_v, rows_v, acc_sh, sem_i, sem_g, sem_s):
    cid = lax.axis_index("c")
    sid = lax.axis_index("s")
    wid = cid * _NS + sid
    r0 = sid * _RPT
    pltpu.sync_copy(zero_hbm.at[pl.ds(r0, _RPT)], acc_sh.at[pl.ds(r0, _RPT)])
    e0 = wid * _EPW

    def idx_copies(c):
        s = lax.rem(c, _NBI)
        return (
            pltpu.make_async_copy(src_hbm.at[pl.ds(e0 + c * _CHUNK, _CHUNK)],
                                  srcs_v.at[s], sem_i.at[s]),
            pltpu.make_async_copy(dst_hbm.at[pl.ds(e0 + c * _CHUNK, _CHUNK)],
                                  dsts_v.at[s], sem_i.at[s]),
        )

    def gather(c):
        s = lax.rem(c, _NBI)
        b = lax.rem(c, _NB)
        return pltpu.make_async_copy(x_hbm.at[srcs_v.at[s]], rows_v.at[b],
                                     sem_g.at[b])

    def scatter(c):
        s = lax.rem(c, _NBI)
        b = lax.rem(c, _NB)
        return pltpu.make_async_copy(rows_v.at[b], acc_sh.at[dsts_v.at[s]],
                                     sem_s.at[b])

    for c in range(2):
        a, d = idx_copies(c)
        a.start()
        d.start()
    plsc.subcore_barrier()

    def step(j, carry):
        @pl.when(j >= 2)
        def _():
            scatter(j - 2).wait()

        @pl.when(j + 2 < _NCHUNK)
        def _():
            a, d = idx_copies(j + 2)
            a.start()
            d.start()

        @pl.when(j < _NCHUNK)
        def _():
            a, d = idx_copies(j)
            a.wait()
            d.wait()
            gather(j).start()

        @pl.when((j >= 1) & (j <= _NCHUNK))
        def _():
            gather(j - 1).wait()
            scatter(j - 1).start(add=True)

        return carry

    lax.fori_loop(0, _NCHUNK + 2, step, 0)
    plsc.subcore_barrier()
    pltpu.sync_copy(acc_sh.at[pl.ds(r0, _RPT)],
                    out_hbm.at[pl.ds(cid * _NP + r0, _RPT)])


_mp = functools.partial(
    pl.kernel,
    out_type=jax.ShapeDtypeStruct((2 * _NP, _D), jnp.float32),
    mesh=plsc.VectorSubcoreMesh(core_axis_name="c", subcore_axis_name="s"),
    scratch_types=[
        pltpu.VMEM((_NBI, _CHUNK), jnp.int32),
        pltpu.VMEM((_NBI, _CHUNK), jnp.int32),
        pltpu.VMEM((_NB, _CHUNK, _D), jnp.float32),
        pltpu.VMEM_SHARED((_NP, _D), jnp.float32),
        pltpu.SemaphoreType.DMA((_NBI,)),
        pltpu.SemaphoreType.DMA((_NB,)),
        pltpu.SemaphoreType.DMA((_NB,)),
    ],
)(_mp_body)


def _dense_body(h_ref, p_ref, w1_ref, b1_ref, g_ref, bt_ref, w2_ref, b2_ref,
                o_ref, *, relu_out):
    h = h_ref[...] + p_ref[:_N] + p_ref[_NP:_NP + _N]
    y = jnp.dot(h, w1_ref[...], preferred_element_type=jnp.float32) + b1_ref[...]
    mu = jnp.mean(y, axis=0, keepdims=True)
    d = y - mu
    var = jnp.mean(d * d, axis=0, keepdims=True)
    r = jnp.maximum(d * lax.rsqrt(var + 1e-5) * g_ref[...] + bt_ref[...], 0.0)
    o = jnp.dot(r, w2_ref[...], preferred_element_type=jnp.float32) + b2_ref[...]
    if relu_out:
        o = jnp.maximum(o, 0.0)
    o_ref[...] = o


def _dense(h, p, w1, b1, g, bt, w2, b2, relu_out):
    return pl.pallas_call(
        functools.partial(_dense_body, relu_out=relu_out),
        out_shape=jax.ShapeDtypeStruct((_N, _D), jnp.float32),
    )(h, p, w1, b1.reshape(1, _D), g.reshape(1, _D), bt.reshape(1, _D),
      w2, b2.reshape(1, _D))


def kernel(x, edge_index, l0_w1, l0_b1, l0_g, l0_beta, l0_w2, l0_b2,
           l1_w1, l1_b1, l1_g, l1_beta, l1_w2, l1_b2):
    src = edge_index[0]
    dst = edge_index[1]
    zeros = jnp.zeros((_NP, _D), jnp.float32)
    p = _mp(x, src, dst, zeros)
    h0 = _dense(x, p, l0_w1, l0_b1, l0_g, l0_beta, l0_w2, l0_b2, relu_out=True)
    p = _mp(h0, src, dst, zeros)
    return _dense(h0, p, l1_w1, l1_b1, l1_g, l1_beta, l1_w2, l1_b2,
                  relu_out=False)


# speedup vs baseline: 11.7860x; 1.0924x over previous
import functools

import jax
import jax.numpy as jnp
from jax import lax
from jax.experimental import pallas as pl
from jax.experimental.pallas import tpu as pltpu
from jax.experimental.pallas import tpu_sc as plsc

_N = 10000
_E = 320000
_D = 128
_NC = 2
_NS = 16
_NW = _NC * _NS
_EPW = _E // _NW
_CHUNK = 40
_NCHUNK = _EPW // _CHUNK
_NB = 4
_NK = 2
_NBI = 8
_NF = _NBI - _NB
_NP = 10112
_RPT = _NP // _NS


def _mp_body(x_hbm, src_hbm, dst_hbm, zero_hbm, out_hbm,
             srcs_v, dsts_v, rows_v, acc_sh, sem_i, sem_g, sem_s):
    cid = lax.axis_index("c")
    sid = lax.axis_index("s")
    wid = cid * _NS + sid
    r0 = sid * _RPT
    pltpu.sync_copy(zero_hbm.at[pl.ds(r0, _RPT)], acc_sh.at[pl.ds(r0, _RPT)])
    e0 = wid * _EPW

    def idx_copies(c):
        s = lax.rem(c, _NBI)
        return (
            pltpu.make_async_copy(src_hbm.at[pl.ds(e0 + c * _CHUNK, _CHUNK)],
                                  srcs_v.at[s], sem_i.at[s]),
            pltpu.make_async_copy(dst_hbm.at[pl.ds(e0 + c * _CHUNK, _CHUNK)],
                                  dsts_v.at[s], sem_i.at[s]),
        )

    def gather(c):
        s = lax.rem(c, _NBI)
        b = lax.rem(c, _NB)
        return pltpu.make_async_copy(x_hbm.at[srcs_v.at[s]], rows_v.at[b],
                                     sem_g.at[b])

    def scatter(c):
        s = lax.rem(c, _NBI)
        b = lax.rem(c, _NB)
        return pltpu.make_async_copy(rows_v.at[b], acc_sh.at[dsts_v.at[s]],
                                     sem_s.at[b])

    for c in range(_NF):
        a, d = idx_copies(c)
        a.start()
        d.start()
    plsc.subcore_barrier()

    def step(j, carry):
        @pl.when(j >= _NB)
        def _():
            scatter(j - _NB).wait()

        @pl.when(j + _NF < _NCHUNK)
        def _():
            a, d = idx_copies(j + _NF)
            a.start()
            d.start()

        @pl.when(j < _NCHUNK)
        def _():
            a, d = idx_copies(j)
            a.wait()
            d.wait()
            gather(j).start()

        @pl.when((j >= _NK) & (j < _NCHUNK + _NK))
        def _():
            gather(j - _NK).wait()
            scatter(j - _NK).start(add=True)

        return carry

    lax.fori_loop(0, _NCHUNK + _NB, step, 0)
    plsc.subcore_barrier()
    pltpu.sync_copy(acc_sh.at[pl.ds(r0, _RPT)],
                    out_hbm.at[pl.ds(cid * _NP + r0, _RPT)])


_mp = functools.partial(
    pl.kernel,
    out_type=jax.ShapeDtypeStruct((2 * _NP, _D), jnp.float32),
    mesh=plsc.VectorSubcoreMesh(core_axis_name="c", subcore_axis_name="s"),
    scratch_types=[
        pltpu.VMEM((_NBI, _CHUNK), jnp.int32),
        pltpu.VMEM((_NBI, _CHUNK), jnp.int32),
        pltpu.VMEM((_NB, _CHUNK, _D), jnp.float32),
        pltpu.VMEM_SHARED((_NP, _D), jnp.float32),
        pltpu.SemaphoreType.DMA((_NBI,)),
        pltpu.SemaphoreType.DMA((_NB,)),
        pltpu.SemaphoreType.DMA((_NB,)),
    ],
)(_mp_body)


def _dense_body(h_ref, p_ref, w1_ref, b1_ref, g_ref, bt_ref, w2_ref, b2_ref,
                o_ref, *, relu_out):
    h = h_ref[...] + p_ref[:_N] + p_ref[_NP:_NP + _N]
    y = jnp.dot(h, w1_ref[...], preferred_element_type=jnp.float32) + b1_ref[...]
    mu = jnp.mean(y, axis=0, keepdims=True)
    d = y - mu
    var = jnp.mean(d * d, axis=0, keepdims=True)
    r = jnp.maximum(d * lax.rsqrt(var + 1e-5) * g_ref[...] + bt_ref[...], 0.0)
    o = jnp.dot(r, w2_ref[...], preferred_element_type=jnp.float32) + b2_ref[...]
    if relu_out:
        o = jnp.maximum(o, 0.0)
    o_ref[...] = o


def _dense(h, p, w1, b1, g, bt, w2, b2, relu_out):
    return pl.pallas_call(
        functools.partial(_dense_body, relu_out=relu_out),
        out_shape=jax.ShapeDtypeStruct((_N, _D), jnp.float32),
    )(h, p, w1, b1.reshape(1, _D), g.reshape(1, _D), bt.reshape(1, _D),
      w2, b2.reshape(1, _D))


def kernel(x, edge_index, l0_w1, l0_b1, l0_g, l0_beta, l0_w2, l0_b2,
           l1_w1, l1_b1, l1_g, l1_beta, l1_w2, l1_b2):
    src = edge_index[0]
    dst = edge_index[1]
    zeros = jnp.zeros((_NP, _D), jnp.float32)
    p = _mp(x, src, dst, zeros)
    h0 = _dense(x, p, l0_w1, l0_b1, l0_g, l0_beta, l0_w2, l0_b2, relu_out=True)
    p = _mp(h0, src, dst, zeros)
    return _dense(h0, p, l1_w1, l1_b1, l1_g, l1_beta, l1_w2, l1_b2,
                  relu_out=False)


# speedup vs baseline: 12.7029x; 1.0778x over previous
import functools

import jax
import jax.numpy as jnp
from jax import lax
from jax.experimental import pallas as pl
from jax.experimental.pallas import tpu as pltpu
from jax.experimental.pallas import tpu_sc as plsc

_N = 10000
_E = 320000
_D = 128
_NC = 2
_NS = 16
_NW = _NC * _NS
_EPW = _E // _NW
_CHUNK = 40
_NCHUNK = _EPW // _CHUNK
_NB = 4
_NK = 3
_NBI = 8
_NF = _NBI - _NB
_NP = 10112
_RPT = _NP // _NS


def _mp_body(x_hbm, src_hbm, dst_hbm, zero_hbm, out_hbm,
             srcs_v, dsts_v, rows_v, acc_sh, sem_i, sem_g, sem_s):
    cid = lax.axis_index("c")
    sid = lax.axis_index("s")
    wid = cid * _NS + sid
    r0 = sid * _RPT
    e0 = wid * _EPW

    def idx_copies(c):
        s = lax.rem(c, _NBI)
        return (
            pltpu.make_async_copy(src_hbm.at[pl.ds(e0 + c * _CHUNK, _CHUNK)],
                                  srcs_v.at[s], sem_i.at[s]),
            pltpu.make_async_copy(dst_hbm.at[pl.ds(e0 + c * _CHUNK, _CHUNK)],
                                  dsts_v.at[s], sem_i.at[s]),
        )

    def gather(c):
        s = lax.rem(c, _NBI)
        b = lax.rem(c, _NB)
        return pltpu.make_async_copy(x_hbm.at[srcs_v.at[s]], rows_v.at[b],
                                     sem_g.at[b])

    def scatter(c):
        s = lax.rem(c, _NBI)
        b = lax.rem(c, _NB)
        return pltpu.make_async_copy(rows_v.at[b], acc_sh.at[dsts_v.at[s]],
                                     sem_s.at[b])

    for c in range(_NF):
        a, d = idx_copies(c)
        a.start()
        d.start()
    pltpu.sync_copy(zero_hbm, rows_v.at[0])
    zcopies = [
        pltpu.make_async_copy(rows_v.at[0],
                              acc_sh.at[pl.ds(r0 + k * _CHUNK, _CHUNK)],
                              sem_s.at[0])
        for k in range(_RPT // _CHUNK)
    ] + [
        pltpu.make_async_copy(rows_v.at[0, pl.ds(0, _RPT % _CHUNK)],
                              acc_sh.at[pl.ds(r0 + _RPT - _RPT % _CHUNK,
                                              _RPT % _CHUNK)],
                              sem_s.at[0])
    ]
    for z in zcopies:
        z.start()
    for z in zcopies:
        z.wait()
    plsc.subcore_barrier()

    def step(j, carry):
        @pl.when(j >= _NB)
        def _():
            scatter(j - _NB).wait()

        @pl.when(j + _NF < _NCHUNK)
        def _():
            a, d = idx_copies(j + _NF)
            a.start()
            d.start()

        @pl.when(j < _NCHUNK)
        def _():
            a, d = idx_copies(j)
            a.wait()
            d.wait()
            gather(j).start()

        @pl.when((j >= _NK) & (j < _NCHUNK + _NK))
        def _():
            gather(j - _NK).wait()
            scatter(j - _NK).start(add=True)

        return carry

    lax.fori_loop(0, _NCHUNK + _NB, step, 0)
    plsc.subcore_barrier()
    pltpu.sync_copy(acc_sh.at[pl.ds(r0, _RPT)],
                    out_hbm.at[pl.ds(cid * _NP + r0, _RPT)])


_mp = functools.partial(
    pl.kernel,
    out_type=jax.ShapeDtypeStruct((2 * _NP, _D), jnp.float32),
    mesh=plsc.VectorSubcoreMesh(core_axis_name="c", subcore_axis_name="s"),
    scratch_types=[
        pltpu.VMEM((_NBI, _CHUNK), jnp.int32),
        pltpu.VMEM((_NBI, _CHUNK), jnp.int32),
        pltpu.VMEM((_NB, _CHUNK, _D), jnp.float32),
        pltpu.VMEM_SHARED((_NP, _D), jnp.float32),
        pltpu.SemaphoreType.DMA((_NBI,)),
        pltpu.SemaphoreType.DMA((_NB,)),
        pltpu.SemaphoreType.DMA((_NB,)),
    ],
)(_mp_body)


def _dense_body(h_ref, p_ref, w1_ref, b1_ref, g_ref, bt_ref, w2_ref, b2_ref,
                o_ref, *, relu_out):
    h = h_ref[...] + p_ref[:_N] + p_ref[_NP:_NP + _N]
    y = jnp.dot(h, w1_ref[...], preferred_element_type=jnp.float32) + b1_ref[...]
    mu = jnp.mean(y, axis=0, keepdims=True)
    d = y - mu
    var = jnp.mean(d * d, axis=0, keepdims=True)
    r = jnp.maximum(d * lax.rsqrt(var + 1e-5) * g_ref[...] + bt_ref[...], 0.0)
    o = jnp.dot(r, w2_ref[...], preferred_element_type=jnp.float32) + b2_ref[...]
    if relu_out:
        o = jnp.maximum(o, 0.0)
    o_ref[...] = o


def _dense(h, p, w1, b1, g, bt, w2, b2, relu_out):
    return pl.pallas_call(
        functools.partial(_dense_body, relu_out=relu_out),
        out_shape=jax.ShapeDtypeStruct((_N, _D), jnp.float32),
    )(h, p, w1, b1.reshape(1, _D), g.reshape(1, _D), bt.reshape(1, _D),
      w2, b2.reshape(1, _D))


def kernel(x, edge_index, l0_w1, l0_b1, l0_g, l0_beta, l0_w2, l0_b2,
           l1_w1, l1_b1, l1_g, l1_beta, l1_w2, l1_b2):
    src = edge_index[0]
    dst = edge_index[1]
    zeros = jnp.zeros((_CHUNK, _D), jnp.float32)
    p = _mp(x, src, dst, zeros)
    h0 = _dense(x, p, l0_w1, l0_b1, l0_g, l0_beta, l0_w2, l0_b2, relu_out=True)
    p = _mp(h0, src, dst, zeros)
    return _dense(h0, p, l1_w1, l1_b1, l1_g, l1_beta, l1_w2, l1_b2,
                  relu_out=False)
